# baseline (speedup 1.0000x reference)
import jax
import jax.numpy as jnp
from jax.experimental import pallas as pl
from jax.experimental.pallas import tpu as pltpu

_C = 64
_QH = 32
_KH = 48
_P = _KH - _QH + 1
_N = _C * _QH * _QH
_LW = _QH * _C
_KW = _KH * _C


def _patch_kernel(q_ref, ka_ref, kb_ref, idx_ref, val_ref):
    p = pl.program_id(0)

    def make_a_body(r):
        def a_body(a, carry):
            best_val, best_idx = carry
            di = a * 8 + r
            accs = [jnp.zeros((8, 128), jnp.float32) for _ in range(_P)]
            for rb in range(4):
                base = pl.multiple_of((a + rb) * 8, 8)
                nrows = 8 if r == 0 else 16
                qb = q_ref[rb * 8:(rb + 1) * 8, :]
                sa = ka_ref[pl.ds(base, nrows), :]
                sb = kb_ref[pl.ds(base, nrows), :]
                if r:
                    sa = jax.lax.slice(sa, (r, 0), (r + 8, _KW))
                    sb = jax.lax.slice(sb, (r, 0), (r + 8, _KW))
                for dj in range(_P):
                    src = sb if (dj % 2) else sa
                    off = (dj // 2) * 128
                    w = jax.lax.slice(src, (0, off), (8, off + _LW))
                    d = jnp.abs(w - qb)
                    for c in range(_LW // 128):
                        accs[dj] = accs[dj] + jax.lax.slice(
                            d, (0, 128 * c), (8, 128 * (c + 1)))
            for dj in range(_P):
                s = jnp.sum(accs[dj])
                idx = di * _P + dj
                take = (s < best_val) | ((s == best_val) & (idx < best_idx))
                best_val = jnp.where(take, s, best_val)
                best_idx = jnp.where(take, idx, best_idx)
            return best_val, best_idx
        return a_body

    def half(rs):
        def run():
            carry = (jnp.float32(jnp.inf), jnp.int32(2**30))
            for r in rs:
                n_a = 3 if r == 0 else 2
                carry = jax.lax.fori_loop(0, n_a, make_a_body(r), carry)
            return carry
        return run

    best_val, best_idx = jax.lax.cond(
        p == 0, half((0, 1, 2, 3)), half((4, 5, 6, 7)))
    idx_ref[p] = best_idx
    val_ref[p] = best_val / jnp.float32(_N)


def kernel(query, key):
    P = int(key.shape[3]) - int(query.shape[3]) + 1

    q = query[0].transpose(1, 2, 0).reshape(_QH, _LW)
    k3 = key[0].transpose(1, 2, 0)
    ka = k3.reshape(_KH, _KW)
    kb = jnp.pad(k3[:, 1:, :], ((0, 0), (0, 1), (0, 0))).reshape(_KH, _KW)

    idxs, vals = pl.pallas_call(
        _patch_kernel,
        grid=(2,),
        in_specs=(
            pl.BlockSpec((_QH, _LW), lambda p: (0, 0)),
            pl.BlockSpec((_KH, _KW), lambda p: (0, 0)),
            pl.BlockSpec((_KH, _KW), lambda p: (0, 0)),
        ),
        out_shape=(
            jax.ShapeDtypeStruct((2,), jnp.int32),
            jax.ShapeDtypeStruct((2,), jnp.float32),
        ),
        out_specs=(
            pl.BlockSpec(memory_space=pltpu.SMEM),
            pl.BlockSpec(memory_space=pltpu.SMEM),
        ),
        compiler_params=pltpu.CompilerParams(
            dimension_semantics=("parallel",)),
    )(q, ka, kb)

    take1 = (vals[1] < vals[0]) | ((vals[1] == vals[0]) & (idxs[1] < idxs[0]))
    hard = jnp.where(take1, idxs[1], idxs[0]).reshape(1)
    rel = jnp.where(take1, vals[1], vals[0]).reshape(1, 1)
    return (hard, P, rel)


# speedup vs baseline: 1.4046x; 1.4046x over previous
import jax
import jax.numpy as jnp
from jax.experimental import pallas as pl
from jax.experimental.pallas import tpu as pltpu

_C = 64
_QH = 32
_KH = 48
_P = _KH - _QH + 1
_N = _C * _QH * _QH
_LW = _QH * _C
_KW = _KH * _C


def _patch_kernel(q_ref, ka_ref, kb_ref, idx_ref, val_ref):

    def make_a_body(r):
        def a_body(a, carry):
            best_val, best_idx = carry
            di = a * 8 + r
            accs = [jnp.zeros((8, 128), jnp.float32) for _ in range(_P)]
            for rb in range(4):
                base = pl.multiple_of((a + rb) * 8, 8)
                nrows = 8 if r == 0 else 16
                qb = q_ref[rb * 8:(rb + 1) * 8, :]
                sa = ka_ref[pl.ds(base, nrows), :]
                sb = kb_ref[pl.ds(base, nrows), :]
                if r:
                    sa = jax.lax.slice(sa, (r, 0), (r + 8, _KW))
                    sb = jax.lax.slice(sb, (r, 0), (r + 8, _KW))
                for dj in range(_P):
                    src = sb if (dj % 2) else sa
                    off = (dj // 2) * 128
                    w = jax.lax.slice(src, (0, off), (8, off + _LW))
                    d = jnp.abs(w - qb)
                    for c in range(_LW // 128):
                        accs[dj] = accs[dj] + jax.lax.slice(
                            d, (0, 128 * c), (8, 128 * (c + 1)))
            for dj in range(_P):
                s = jnp.sum(accs[dj])
                idx = di * _P + dj
                take = (s < best_val) | ((s == best_val) & (idx < best_idx))
                best_val = jnp.where(take, s, best_val)
                best_idx = jnp.where(take, idx, best_idx)
            return best_val, best_idx
        return a_body

    carry = (jnp.float32(jnp.inf), jnp.int32(2**30))
    for r in range(8):
        n_a = 3 if r == 0 else 2
        carry = jax.lax.fori_loop(0, n_a, make_a_body(r), carry)
    best_val, best_idx = carry
    idx_ref[0] = best_idx
    val_ref[0, 0] = best_val / jnp.float32(_N)


def kernel(query, key):
    P = int(key.shape[3]) - int(query.shape[3]) + 1

    q = query[0].transpose(1, 2, 0).reshape(_QH, _LW)
    k3 = key[0].transpose(1, 2, 0)
    ka = k3.reshape(_KH, _KW)
    kb = jnp.pad(k3[:, 1:, :], ((0, 0), (0, 1), (0, 0))).reshape(_KH, _KW)

    idx, val = pl.pallas_call(
        _patch_kernel,
        out_shape=(
            jax.ShapeDtypeStruct((1,), jnp.int32),
            jax.ShapeDtypeStruct((1, 1), jnp.float32),
        ),
        out_specs=(
            pl.BlockSpec(memory_space=pltpu.SMEM),
            pl.BlockSpec(memory_space=pltpu.SMEM),
        ),
        compiler_params=pltpu.CompilerParams(
            allow_input_fusion=(True, True, True)),
    )(q, ka, kb)

    return (idx, P, val)
